# baseline (speedup 1.0000x reference)
import functools

import jax
import jax.numpy as jnp
from jax import lax
from jax.experimental import pallas as pl
from jax.experimental.pallas import tpu as pltpu
from jax.experimental.pallas import tpu_sc as plsc

DMODEL = 1024
H = DMODEL // 2
DQK = 64
HEADS = 16
HD = HEADS * DQK
FF = 2048
NB = 64
PEN = 100000.0
D = 2
N = 4096
T = D * N
CH = 128
BC = 4
NCHUNK = T // CH

_F32 = jnp.float32


HBLK = 512
NHSTEP = T // HBLK


def _hash_sort_body(x_ref, wqk_ref, bqk_ref, hm_ref, x2_ref, p_ref, g_ref,
                    aux_ref, h_acc):
    i = pl.program_id(0)

    @pl.when(i < NHSTEP)
    def _hash():
        x2 = x_ref[...]
        x2_ref[...] = x2
        qk = jnp.dot(x2, wqk_ref[...],
                     preferred_element_type=_F32) + bqk_ref[...]
        proj = jnp.dot(qk, hm_ref[...], preferred_element_type=_F32)
        m = jnp.maximum(jnp.max(proj, axis=1, keepdims=True),
                        jnp.max(-proj, axis=1, keepdims=True))
        io = lax.broadcasted_iota(jnp.int32, (HBLK, NB // 2), 1)
        big = jnp.int32(2 * NB)
        i1 = jnp.min(jnp.where(proj == m, io, big), axis=1, keepdims=True)
        i2 = jnp.min(jnp.where(-proj == m, io + (NB // 2), big), axis=1,
                     keepdims=True)
        h_acc[pl.ds(i * HBLK, HBLK), :] = jnp.minimum(i1, i2)

    @pl.when(i == NHSTEP)
    def _gram():
        w = wqk_ref[...]
        g_ref[...] = lax.dot_general(w, w, (((1,), (1,)), ((), ())),
                                     preferred_element_type=_F32)
        bq = bqk_ref[...]
        wb = lax.dot_general(bq, w, (((1,), (1,)), ((), ())),
                             preferred_element_type=_F32)
        aux_ref[0:1, :] = wb
        aux_ref[1:2, :] = jnp.broadcast_to(
            jnp.sum(bq * bq, axis=1, keepdims=True), (1, H))

    @pl.when(i >= NHSTEP)
    def _sort():
        b = i - NHSTEP
        h = h_acc[pl.ds(b * N, N), :]
        buckets = lax.broadcasted_iota(jnp.int32, (N, NB), 1)
        onehot = (h == buckets).astype(_F32)
        counts = jnp.sum(onehot, axis=0, keepdims=True)
        r0 = lax.broadcasted_iota(jnp.int32, (NB, NB), 0)
        c0 = lax.broadcasted_iota(jnp.int32, (NB, NB), 1)
        lt = (r0 < c0).astype(_F32)
        starts = jnp.dot(counts, lt, preferred_element_type=_F32)
        tb = 512
        nblk = N // tb
        r1 = lax.broadcasted_iota(jnp.int32, (tb, tb), 0)
        c1 = lax.broadcasted_iota(jnp.int32, (tb, tb), 1)
        lstrict = (c1 < r1).astype(_F32)
        run = starts
        for j in range(nblk):
            ob = lax.slice(onehot, (j * tb, 0), ((j + 1) * tb, NB))
            cb = jnp.dot(lstrict, ob, preferred_element_type=_F32) + run
            pb = jnp.sum(cb * ob, axis=1, keepdims=True)
            run = run + jnp.sum(ob, axis=0, keepdims=True)
            p_ref[0, j, :] = pb[:, 0].astype(jnp.int32) + b * N


def _hash_and_sort(xf, wqk, bqk, hm):
    nstep = NHSTEP + D
    x2f, p, gmat, aux = pl.pallas_call(
        _hash_sort_body,
        grid=(nstep,),
        in_specs=[
            pl.BlockSpec((HBLK, H), lambda i: (jnp.minimum(i, NHSTEP - 1), 1)),
            pl.BlockSpec((H, HD), lambda i: (0, 0)),
            pl.BlockSpec((1, HD), lambda i: (0, 0)),
            pl.BlockSpec((HD, NB // 2), lambda i: (0, 0)),
        ],
        out_specs=[
            pl.BlockSpec((HBLK, H), lambda i: (jnp.minimum(i, NHSTEP - 1), 0)),
            pl.BlockSpec((1, N // 512, 512),
                         lambda i: (jnp.maximum(i - NHSTEP, 0), 0, 0)),
            pl.BlockSpec((H, H), lambda i: (0, 0)),
            pl.BlockSpec((8, H), lambda i: (0, 0)),
        ],
        out_shape=[
            jax.ShapeDtypeStruct((T, H), _F32),
            jax.ShapeDtypeStruct((D, N // 512, 512), jnp.int32),
            jax.ShapeDtypeStruct((H, H), _F32),
            jax.ShapeDtypeStruct((8, H), _F32),
        ],
        scratch_shapes=[pltpu.VMEM((T, 1), jnp.int32)],
        compiler_params=pltpu.CompilerParams(
            dimension_semantics=("arbitrary",)),
    )(xf, wqk, bqk, hm)
    return x2f, p.reshape(T), gmat, aux


RC = 64


def _sc_permute(x2f, pos2d):
    info = plsc.get_sparse_core_info()
    nw = info.num_cores * info.num_subcores
    per_w = T // nw
    nchunk = per_w // RC
    mesh = plsc.VectorSubcoreMesh(core_axis_name="c", subcore_axis_name="s")

    @functools.partial(
        pl.kernel,
        mesh=mesh,
        out_type=jax.ShapeDtypeStruct((T, H), _F32),
        scratch_types=[
            pltpu.VMEM((nchunk, RC), jnp.int32),
            pltpu.VMEM((RC, H), _F32),
            pltpu.VMEM((RC, H), _F32),
            pltpu.VMEM((RC, H), _F32),
            pltpu.SemaphoreType.DMA,
            pltpu.SemaphoreType.DMA,
            pltpu.SemaphoreType.DMA,
            pltpu.SemaphoreType.DMA,
            pltpu.SemaphoreType.DMA,
            pltpu.SemaphoreType.DMA,
        ],
    )
    def scatter_k(tab_hbm, idx_hbm, out_hbm, idx_v, b0, b1, b2,
                  l0, l1, l2, s0, s1, s2):
        wid = lax.axis_index("s") * info.num_cores + lax.axis_index("c")
        base = wid * per_w
        pltpu.sync_copy(idx_hbm.at[pl.ds(wid * nchunk, nchunk)], idx_v)
        bufs = [b0, b1, b2]
        lsems = [l0, l1, l2]
        ssems = [s0, s1, s2]

        def load(j, b):
            return pltpu.async_copy(tab_hbm.at[pl.ds(base + j * RC, RC)],
                                    bufs[b], lsems[b])

        def scat(j, b):
            return pltpu.async_copy(bufs[b], out_hbm.at[idx_v.at[j]],
                                    ssems[b])

        loads = {0: load(0, 0), 1: load(1, 1)}
        scats = {}
        for j in range(nchunk):
            b = j % 3
            loads[j].wait()
            scats[j] = scat(j, b)
            nj = j + 2
            if nj < nchunk:
                if nj - 3 >= 0:
                    scats[nj - 3].wait()
                loads[nj] = load(nj, nj % 3)
        for j in range(max(nchunk - 3, 1), nchunk):
            scats[j].wait()

    return scatter_k(x2f, pos2d)
def _tail_body(x_ref, x2g_ref, g_ref, aux_ref, wv_ref, bv_ref,
               wu_ref, bu_ref, n1g_ref, n1b_ref,
               w1_ref, b1_ref, w2_ref, b2_ref, n2g_ref, n2b_ref, o_ref):
    x2g = x2g_ref[...]
    valsf = jnp.dot(x2g, wv_ref[...], preferred_element_type=_F32) + bv_ref[...]
    xg = jnp.dot(x2g, g_ref[...], preferred_element_type=_F32)
    wb = aux_ref[0:1, :]
    u = lax.dot_general(x2g, wb, (((1,), (1,)), ((), ())),
                        preferred_element_type=_F32)
    bb = aux_ref[1:2, 0:1]
    ones_col = jnp.ones((CH, 1), _F32)
    rr = lax.broadcasted_iota(jnp.int32, (CH, CH), 0)
    cc = lax.broadcasted_iota(jnp.int32, (CH, CH), 1)
    parts = []
    for sub in range(BC):
        xgs = lax.slice(xg, (sub * CH, 0), ((sub + 1) * CH, H))
        xs = lax.slice(x2g, (sub * CH, 0), ((sub + 1) * CH, H))
        us = lax.slice(u, (sub * CH, 0), ((sub + 1) * CH, 1))
        v = lax.slice(valsf, (sub * CH, 0), ((sub + 1) * CH, HD))
        s = lax.dot_general(xgs, xs, (((1,), (1,)), ((), ())),
                            preferred_element_type=_F32)
        urow = lax.dot_general(ones_col, us, (((1,), (1,)), ((), ())),
                               preferred_element_type=_F32)
        s = (s + us + urow + bb) * _F32(1.0 / 8.0)
        s = jnp.where(rr == cc, s / _F32(PEN), s)
        parts.append(jnp.dot(s, v, preferred_element_type=_F32))
    attn = jnp.concatenate(parts, axis=0)
    ua = jnp.dot(attn, wu_ref[...], preferred_element_type=_F32) + bu_ref[...]

    x1 = x_ref[:, :H]
    x2 = x_ref[:, H:]
    t1 = x1 + ua
    mu = jnp.mean(t1, axis=1, keepdims=True)
    var = jnp.mean((t1 - mu) ** 2, axis=1, keepdims=True)
    y1 = (t1 - mu) / jnp.sqrt(var + _F32(1e-5)) * n1g_ref[...] + n1b_ref[...]

    ff = jnp.maximum(jnp.dot(y1, w1_ref[...], preferred_element_type=_F32)
                     + b1_ref[...], _F32(0.0))
    ffo = jnp.dot(ff, w2_ref[...], preferred_element_type=_F32) + b2_ref[...]
    t2 = x2 + ffo
    mu2 = jnp.mean(t2, axis=1, keepdims=True)
    var2 = jnp.mean((t2 - mu2) ** 2, axis=1, keepdims=True)
    y2 = (t2 - mu2) / jnp.sqrt(var2 + _F32(1e-5)) * n2g_ref[...] + n2b_ref[...]

    o_ref[:, :H] = y1
    o_ref[:, H:] = y2


def _attention_tail(xf, x2g, gmat, aux, wv, bv, wu, bu, n1g, n1b,
                    w1, b1, w2, b2, n2g, n2b):
    res = lambda shape: pl.BlockSpec(shape, lambda c: tuple(0 for _ in shape))
    return pl.pallas_call(
        _tail_body,
        grid=(NCHUNK // BC,),
        in_specs=[
            pl.BlockSpec((BC * CH, DMODEL), lambda c: (c, 0)),
            pl.BlockSpec((BC * CH, H), lambda c: (c, 0)),
            res((H, H)), res((8, H)),
            res((H, HD)), res((1, HD)),
            res((HD, H)), res((1, H)),
            res((1, H)), res((1, H)),
            res((H, FF)), res((1, FF)),
            res((FF, H)), res((1, H)),
            res((1, H)), res((1, H)),
        ],
        out_specs=pl.BlockSpec((BC * CH, DMODEL), lambda c: (c, 0)),
        out_shape=jax.ShapeDtypeStruct((T, DMODEL), _F32),
    )(xf, x2g, gmat, aux, wv, bv, wu, bu, n1g, n1b, w1, b1, w2, b2, n2g, n2b)


def kernel(x, Wqk_w, Wqk_b, Wv_w, Wv_b, unify_w, unify_b, n1_g, n1_b,
           ff_w1, ff_b1, ff_w2, ff_b2, n2_g, n2_b, hashM):
    xf = x.reshape(T, DMODEL)
    r = lambda v: v.reshape(1, -1)
    x2f, pos, gmat, aux = _hash_and_sort(xf, Wqk_w, r(Wqk_b), hashM)
    x2g = _sc_permute(x2f, pos.reshape(T // RC, RC))
    out = _attention_tail(xf, x2g, gmat, aux, Wv_w, r(Wv_b),
                          unify_w, r(unify_b), r(n1_g), r(n1_b),
                          ff_w1, r(ff_b1), ff_w2, r(ff_b2), r(n2_g), r(n2_b))
    return out.reshape(D, N, DMODEL)


# speedup vs baseline: 1.0372x; 1.0372x over previous
import functools

import jax
import jax.numpy as jnp
from jax import lax
from jax.experimental import pallas as pl
from jax.experimental.pallas import tpu as pltpu
from jax.experimental.pallas import tpu_sc as plsc

DMODEL = 1024
H = DMODEL // 2
DQK = 64
HEADS = 16
HD = HEADS * DQK
FF = 2048
NB = 64
PEN = 100000.0
D = 2
N = 4096
T = D * N
CH = 128
BC = 4
NCHUNK = T // CH

_F32 = jnp.float32


HBLK = 512
NHSTEP = T // HBLK


def _hash_sort_body(x_ref, wqk_ref, bqk_ref, hm_ref, x2_ref, p_ref, h_acc):
    i = pl.program_id(0)

    @pl.when(i < NHSTEP)
    def _hash():
        x2 = x_ref[...]
        x2_ref[...] = x2
        qk = jnp.dot(x2, wqk_ref[...],
                     preferred_element_type=_F32) + bqk_ref[...]
        proj = jnp.dot(qk, hm_ref[...], preferred_element_type=_F32)
        m = jnp.maximum(jnp.max(proj, axis=1, keepdims=True),
                        jnp.max(-proj, axis=1, keepdims=True))
        io = lax.broadcasted_iota(jnp.int32, (HBLK, NB // 2), 1)
        big = jnp.int32(2 * NB)
        i1 = jnp.min(jnp.where(proj == m, io, big), axis=1, keepdims=True)
        i2 = jnp.min(jnp.where(-proj == m, io + (NB // 2), big), axis=1,
                     keepdims=True)
        h_acc[pl.ds(i * HBLK, HBLK), :] = jnp.minimum(i1, i2)

    @pl.when(i >= NHSTEP)
    def _sort():
        b = i - NHSTEP
        h = h_acc[pl.ds(b * N, N), :]
        buckets = lax.broadcasted_iota(jnp.int32, (N, NB), 1)
        onehot = (h == buckets).astype(_F32)
        counts = jnp.sum(onehot, axis=0, keepdims=True)
        r0 = lax.broadcasted_iota(jnp.int32, (NB, NB), 0)
        c0 = lax.broadcasted_iota(jnp.int32, (NB, NB), 1)
        lt = (r0 < c0).astype(_F32)
        starts = jnp.dot(counts, lt, preferred_element_type=_F32)
        tb = 512
        nblk = N // tb
        r1 = lax.broadcasted_iota(jnp.int32, (tb, tb), 0)
        c1 = lax.broadcasted_iota(jnp.int32, (tb, tb), 1)
        lstrict = (c1 < r1).astype(_F32)
        run = starts
        for j in range(nblk):
            ob = lax.slice(onehot, (j * tb, 0), ((j + 1) * tb, NB))
            cb = jnp.dot(lstrict, ob, preferred_element_type=_F32) + run
            pb = jnp.sum(cb * ob, axis=1, keepdims=True)
            run = run + jnp.sum(ob, axis=0, keepdims=True)
            p_ref[0, j, :] = pb[:, 0].astype(jnp.int32) + b * N


def _hash_and_sort(xf, wqk, bqk, hm):
    nstep = NHSTEP + D
    x2f, p = pl.pallas_call(
        _hash_sort_body,
        grid=(nstep,),
        in_specs=[
            pl.BlockSpec((HBLK, H), lambda i: (jnp.minimum(i, NHSTEP - 1), 1)),
            pl.BlockSpec((H, HD), lambda i: (0, 0)),
            pl.BlockSpec((1, HD), lambda i: (0, 0)),
            pl.BlockSpec((HD, NB // 2), lambda i: (0, 0)),
        ],
        out_specs=[
            pl.BlockSpec((HBLK, H), lambda i: (jnp.minimum(i, NHSTEP - 1), 0)),
            pl.BlockSpec((1, N // 512, 512),
                         lambda i: (jnp.maximum(i - NHSTEP, 0), 0, 0)),
        ],
        out_shape=[
            jax.ShapeDtypeStruct((T, H), _F32),
            jax.ShapeDtypeStruct((D, N // 512, 512), jnp.int32),
        ],
        scratch_shapes=[pltpu.VMEM((T, 1), jnp.int32)],
        compiler_params=pltpu.CompilerParams(
            dimension_semantics=("arbitrary",)),
    )(xf, wqk, bqk, hm)
    return x2f, p.reshape(T)


RC = 64


def _sc_permute(x2f, pos2d):
    info = plsc.get_sparse_core_info()
    nw = info.num_cores * info.num_subcores
    per_w = T // nw
    nchunk = per_w // RC
    mesh = plsc.VectorSubcoreMesh(core_axis_name="c", subcore_axis_name="s")

    @functools.partial(
        pl.kernel,
        mesh=mesh,
        out_type=jax.ShapeDtypeStruct((T, H), _F32),
        scratch_types=[
            pltpu.VMEM((nchunk, RC), jnp.int32),
            pltpu.VMEM((RC, H), _F32),
            pltpu.VMEM((RC, H), _F32),
            pltpu.VMEM((RC, H), _F32),
            pltpu.SemaphoreType.DMA,
            pltpu.SemaphoreType.DMA,
            pltpu.SemaphoreType.DMA,
            pltpu.SemaphoreType.DMA,
            pltpu.SemaphoreType.DMA,
            pltpu.SemaphoreType.DMA,
        ],
    )
    def scatter_k(tab_hbm, idx_hbm, out_hbm, idx_v, b0, b1, b2,
                  l0, l1, l2, s0, s1, s2):
        wid = lax.axis_index("s") * info.num_cores + lax.axis_index("c")
        base = wid * per_w
        pltpu.sync_copy(idx_hbm.at[pl.ds(wid * nchunk, nchunk)], idx_v)
        bufs = [b0, b1, b2]
        lsems = [l0, l1, l2]
        ssems = [s0, s1, s2]

        def load(j, b):
            return pltpu.async_copy(tab_hbm.at[pl.ds(base + j * RC, RC)],
                                    bufs[b], lsems[b])

        def scat(j, b):
            return pltpu.async_copy(bufs[b], out_hbm.at[idx_v.at[j]],
                                    ssems[b])

        loads = {0: load(0, 0), 1: load(1, 1)}
        scats = {}
        for j in range(nchunk):
            b = j % 3
            loads[j].wait()
            scats[j] = scat(j, b)
            nj = j + 2
            if nj < nchunk:
                if nj - 3 >= 0:
                    scats[nj - 3].wait()
                loads[nj] = load(nj, nj % 3)
        for j in range(max(nchunk - 3, 1), nchunk):
            scats[j].wait()

    return scatter_k(x2f, pos2d)
def _tail_body(x_ref, x2g_ref, wqk_ref, bqk_ref, wv_ref, bv_ref,
               wu_ref, bu_ref, n1g_ref, n1b_ref,
               w1_ref, b1_ref, w2_ref, b2_ref, n2g_ref, n2b_ref, o_ref):
    x2g = x2g_ref[...]
    qkf = jnp.dot(x2g, wqk_ref[...], preferred_element_type=_F32) + bqk_ref[...]
    valsf = jnp.dot(x2g, wv_ref[...], preferred_element_type=_F32) + bv_ref[...]
    rr = lax.broadcasted_iota(jnp.int32, (CH, CH), 0)
    cc = lax.broadcasted_iota(jnp.int32, (CH, CH), 1)
    parts = []
    for sub in range(BC):
        q = lax.slice(qkf, (sub * CH, 0), ((sub + 1) * CH, HD))
        v = lax.slice(valsf, (sub * CH, 0), ((sub + 1) * CH, HD))
        s = lax.dot_general(q, q, (((1,), (1,)), ((), ())),
                            preferred_element_type=_F32) * _F32(1.0 / 8.0)
        s = jnp.where(rr == cc, s / _F32(PEN), s)
        parts.append(jnp.dot(s, v, preferred_element_type=_F32))
    attn = jnp.concatenate(parts, axis=0)
    ua = jnp.dot(attn, wu_ref[...], preferred_element_type=_F32) + bu_ref[...]

    x1 = x_ref[:, :H]
    x2 = x_ref[:, H:]
    t1 = x1 + ua
    mu = jnp.mean(t1, axis=1, keepdims=True)
    var = jnp.mean((t1 - mu) ** 2, axis=1, keepdims=True)
    y1 = (t1 - mu) / jnp.sqrt(var + _F32(1e-5)) * n1g_ref[...] + n1b_ref[...]

    ff = jnp.maximum(jnp.dot(y1, w1_ref[...], preferred_element_type=_F32)
                     + b1_ref[...], _F32(0.0))
    ffo = jnp.dot(ff, w2_ref[...], preferred_element_type=_F32) + b2_ref[...]
    t2 = x2 + ffo
    mu2 = jnp.mean(t2, axis=1, keepdims=True)
    var2 = jnp.mean((t2 - mu2) ** 2, axis=1, keepdims=True)
    y2 = (t2 - mu2) / jnp.sqrt(var2 + _F32(1e-5)) * n2g_ref[...] + n2b_ref[...]

    o_ref[:, :H] = y1
    o_ref[:, H:] = y2


def _attention_tail(xf, x2g, wqk, bqk, wv, bv, wu, bu, n1g, n1b,
                    w1, b1, w2, b2, n2g, n2b):
    res = lambda shape: pl.BlockSpec(shape, lambda c: tuple(0 for _ in shape))
    return pl.pallas_call(
        _tail_body,
        grid=(NCHUNK // BC,),
        in_specs=[
            pl.BlockSpec((BC * CH, DMODEL), lambda c: (c, 0)),
            pl.BlockSpec((BC * CH, H), lambda c: (c, 0)),
            res((H, HD)), res((1, HD)),
            res((H, HD)), res((1, HD)),
            res((HD, H)), res((1, H)),
            res((1, H)), res((1, H)),
            res((H, FF)), res((1, FF)),
            res((FF, H)), res((1, H)),
            res((1, H)), res((1, H)),
        ],
        out_specs=pl.BlockSpec((BC * CH, DMODEL), lambda c: (c, 0)),
        out_shape=jax.ShapeDtypeStruct((T, DMODEL), _F32),
    )(xf, x2g, wqk, bqk, wv, bv, wu, bu, n1g, n1b, w1, b1, w2, b2, n2g, n2b)


def kernel(x, Wqk_w, Wqk_b, Wv_w, Wv_b, unify_w, unify_b, n1_g, n1_b,
           ff_w1, ff_b1, ff_w2, ff_b2, n2_g, n2_b, hashM):
    xf = x.reshape(T, DMODEL)
    r = lambda v: v.reshape(1, -1)
    x2f, pos = _hash_and_sort(xf, Wqk_w, r(Wqk_b), hashM)
    x2g = _sc_permute(x2f, pos.reshape(T // RC, RC))
    out = _attention_tail(xf, x2g, Wqk_w, r(Wqk_b), Wv_w, r(Wv_b),
                          unify_w, r(unify_b), r(n1_g), r(n1_b),
                          ff_w1, r(ff_b1), ff_w2, r(ff_b2), r(n2_g), r(n2_b))
    return out.reshape(D, N, DMODEL)


# speedup vs baseline: 1.0743x; 1.0358x over previous
import functools

import jax
import jax.numpy as jnp
from jax import lax
from jax.experimental import pallas as pl
from jax.experimental.pallas import tpu as pltpu
from jax.experimental.pallas import tpu_sc as plsc

DMODEL = 1024
H = DMODEL // 2
DQK = 64
HEADS = 16
HD = HEADS * DQK
FF = 2048
NB = 64
PEN = 100000.0
D = 2
N = 4096
T = D * N
CH = 128
BC = 8
NCHUNK = T // CH

_F32 = jnp.float32


HBLK = 512
NHSTEP = T // HBLK


def _hash_sort_body(x_ref, wqk_ref, bqk_ref, hm_ref, x2_ref, p_ref, h_acc):
    i = pl.program_id(0)

    @pl.when(i < NHSTEP)
    def _hash():
        x2 = x_ref[...]
        x2_ref[...] = x2
        qk = jnp.dot(x2, wqk_ref[...],
                     preferred_element_type=_F32) + bqk_ref[...]
        proj = jnp.dot(qk, hm_ref[...], preferred_element_type=_F32)
        m = jnp.maximum(jnp.max(proj, axis=1, keepdims=True),
                        jnp.max(-proj, axis=1, keepdims=True))
        io = lax.broadcasted_iota(jnp.int32, (HBLK, NB // 2), 1)
        big = jnp.int32(2 * NB)
        i1 = jnp.min(jnp.where(proj == m, io, big), axis=1, keepdims=True)
        i2 = jnp.min(jnp.where(-proj == m, io + (NB // 2), big), axis=1,
                     keepdims=True)
        h_acc[pl.ds(i * HBLK, HBLK), :] = jnp.minimum(i1, i2)

    @pl.when(i >= NHSTEP)
    def _sort():
        b = i - NHSTEP
        h = h_acc[pl.ds(b * N, N), :]
        buckets = lax.broadcasted_iota(jnp.int32, (N, NB), 1)
        onehot = (h == buckets).astype(_F32)
        counts = jnp.sum(onehot, axis=0, keepdims=True)
        r0 = lax.broadcasted_iota(jnp.int32, (NB, NB), 0)
        c0 = lax.broadcasted_iota(jnp.int32, (NB, NB), 1)
        lt = (r0 < c0).astype(_F32)
        starts = jnp.dot(counts, lt, preferred_element_type=_F32)
        tb = 512
        nblk = N // tb
        r1 = lax.broadcasted_iota(jnp.int32, (tb, tb), 0)
        c1 = lax.broadcasted_iota(jnp.int32, (tb, tb), 1)
        lstrict = (c1 < r1).astype(_F32)
        run = starts
        for j in range(nblk):
            ob = lax.slice(onehot, (j * tb, 0), ((j + 1) * tb, NB))
            cb = jnp.dot(lstrict, ob, preferred_element_type=_F32) + run
            pb = jnp.sum(cb * ob, axis=1, keepdims=True)
            run = run + jnp.sum(ob, axis=0, keepdims=True)
            p_ref[0, j, :] = pb[:, 0].astype(jnp.int32) + b * N


def _hash_and_sort(xf, wqk, bqk, hm):
    nstep = NHSTEP + D
    x2f, p = pl.pallas_call(
        _hash_sort_body,
        grid=(nstep,),
        in_specs=[
            pl.BlockSpec((HBLK, H), lambda i: (jnp.minimum(i, NHSTEP - 1), 1)),
            pl.BlockSpec((H, HD), lambda i: (0, 0)),
            pl.BlockSpec((1, HD), lambda i: (0, 0)),
            pl.BlockSpec((HD, NB // 2), lambda i: (0, 0)),
        ],
        out_specs=[
            pl.BlockSpec((HBLK, H), lambda i: (jnp.minimum(i, NHSTEP - 1), 0)),
            pl.BlockSpec((1, N // 512, 512),
                         lambda i: (jnp.maximum(i - NHSTEP, 0), 0, 0)),
        ],
        out_shape=[
            jax.ShapeDtypeStruct((T, H), _F32),
            jax.ShapeDtypeStruct((D, N // 512, 512), jnp.int32),
        ],
        scratch_shapes=[pltpu.VMEM((T, 1), jnp.int32)],
        compiler_params=pltpu.CompilerParams(
            dimension_semantics=("arbitrary",)),
    )(xf, wqk, bqk, hm)
    return x2f, p.reshape(T)


RC = 64


def _sc_permute(x2f, pos2d):
    info = plsc.get_sparse_core_info()
    nw = info.num_cores * info.num_subcores
    per_w = T // nw
    nchunk = per_w // RC
    mesh = plsc.VectorSubcoreMesh(core_axis_name="c", subcore_axis_name="s")

    @functools.partial(
        pl.kernel,
        mesh=mesh,
        out_type=jax.ShapeDtypeStruct((T, H), _F32),
        scratch_types=[
            pltpu.VMEM((nchunk, RC), jnp.int32),
            pltpu.VMEM((RC, H), _F32),
            pltpu.VMEM((RC, H), _F32),
            pltpu.VMEM((RC, H), _F32),
            pltpu.SemaphoreType.DMA,
            pltpu.SemaphoreType.DMA,
            pltpu.SemaphoreType.DMA,
            pltpu.SemaphoreType.DMA,
            pltpu.SemaphoreType.DMA,
            pltpu.SemaphoreType.DMA,
        ],
    )
    def scatter_k(tab_hbm, idx_hbm, out_hbm, idx_v, b0, b1, b2,
                  l0, l1, l2, s0, s1, s2):
        wid = lax.axis_index("s") * info.num_cores + lax.axis_index("c")
        base = wid * per_w
        pltpu.sync_copy(idx_hbm.at[pl.ds(wid * nchunk, nchunk)], idx_v)
        bufs = [b0, b1, b2]
        lsems = [l0, l1, l2]
        ssems = [s0, s1, s2]

        def load(j, b):
            return pltpu.async_copy(tab_hbm.at[pl.ds(base + j * RC, RC)],
                                    bufs[b], lsems[b])

        def scat(j, b):
            return pltpu.async_copy(bufs[b], out_hbm.at[idx_v.at[j]],
                                    ssems[b])

        loads = {0: load(0, 0), 1: load(1, 1)}
        scats = {}
        for j in range(nchunk):
            b = j % 3
            loads[j].wait()
            scats[j] = scat(j, b)
            nj = j + 2
            if nj < nchunk:
                if nj - 3 >= 0:
                    scats[nj - 3].wait()
                loads[nj] = load(nj, nj % 3)
        for j in range(max(nchunk - 3, 1), nchunk):
            scats[j].wait()

    return scatter_k(x2f, pos2d)
def _tail_body(x_ref, x2g_ref, wqk_ref, bqk_ref, wv_ref, bv_ref,
               wu_ref, bu_ref, n1g_ref, n1b_ref,
               w1_ref, b1_ref, w2_ref, b2_ref, n2g_ref, n2b_ref, o_ref):
    x2g = x2g_ref[...]
    qkf = jnp.dot(x2g, wqk_ref[...], preferred_element_type=_F32) + bqk_ref[...]
    valsf = jnp.dot(x2g, wv_ref[...], preferred_element_type=_F32) + bv_ref[...]
    rr = lax.broadcasted_iota(jnp.int32, (CH, CH), 0)
    cc = lax.broadcasted_iota(jnp.int32, (CH, CH), 1)
    parts = []
    for sub in range(BC):
        q = lax.slice(qkf, (sub * CH, 0), ((sub + 1) * CH, HD))
        v = lax.slice(valsf, (sub * CH, 0), ((sub + 1) * CH, HD))
        s = lax.dot_general(q, q, (((1,), (1,)), ((), ())),
                            preferred_element_type=_F32) * _F32(1.0 / 8.0)
        s = jnp.where(rr == cc, s / _F32(PEN), s)
        parts.append(jnp.dot(s, v, preferred_element_type=_F32))
    attn = jnp.concatenate(parts, axis=0)
    ua = jnp.dot(attn, wu_ref[...], preferred_element_type=_F32) + bu_ref[...]

    x1 = x_ref[:, :H]
    x2 = x_ref[:, H:]
    t1 = x1 + ua
    mu = jnp.mean(t1, axis=1, keepdims=True)
    var = jnp.mean((t1 - mu) ** 2, axis=1, keepdims=True)
    y1 = (t1 - mu) / jnp.sqrt(var + _F32(1e-5)) * n1g_ref[...] + n1b_ref[...]

    ff = jnp.maximum(jnp.dot(y1, w1_ref[...], preferred_element_type=_F32)
                     + b1_ref[...], _F32(0.0))
    ffo = jnp.dot(ff, w2_ref[...], preferred_element_type=_F32) + b2_ref[...]
    t2 = x2 + ffo
    mu2 = jnp.mean(t2, axis=1, keepdims=True)
    var2 = jnp.mean((t2 - mu2) ** 2, axis=1, keepdims=True)
    y2 = (t2 - mu2) / jnp.sqrt(var2 + _F32(1e-5)) * n2g_ref[...] + n2b_ref[...]

    o_ref[:, :H] = y1
    o_ref[:, H:] = y2


def _attention_tail(xf, x2g, wqk, bqk, wv, bv, wu, bu, n1g, n1b,
                    w1, b1, w2, b2, n2g, n2b):
    res = lambda shape: pl.BlockSpec(shape, lambda c: tuple(0 for _ in shape))
    return pl.pallas_call(
        _tail_body,
        grid=(NCHUNK // BC,),
        in_specs=[
            pl.BlockSpec((BC * CH, DMODEL), lambda c: (c, 0)),
            pl.BlockSpec((BC * CH, H), lambda c: (c, 0)),
            res((H, HD)), res((1, HD)),
            res((H, HD)), res((1, HD)),
            res((HD, H)), res((1, H)),
            res((1, H)), res((1, H)),
            res((H, FF)), res((1, FF)),
            res((FF, H)), res((1, H)),
            res((1, H)), res((1, H)),
        ],
        out_specs=pl.BlockSpec((BC * CH, DMODEL), lambda c: (c, 0)),
        out_shape=jax.ShapeDtypeStruct((T, DMODEL), _F32),
    )(xf, x2g, wqk, bqk, wv, bv, wu, bu, n1g, n1b, w1, b1, w2, b2, n2g, n2b)


def kernel(x, Wqk_w, Wqk_b, Wv_w, Wv_b, unify_w, unify_b, n1_g, n1_b,
           ff_w1, ff_b1, ff_w2, ff_b2, n2_g, n2_b, hashM):
    xf = x.reshape(T, DMODEL)
    r = lambda v: v.reshape(1, -1)
    x2f, pos = _hash_and_sort(xf, Wqk_w, r(Wqk_b), hashM)
    x2g = _sc_permute(x2f, pos.reshape(T // RC, RC))
    out = _attention_tail(xf, x2g, Wqk_w, r(Wqk_b), Wv_w, r(Wv_b),
                          unify_w, r(unify_b), r(n1_g), r(n1_b),
                          ff_w1, r(ff_b1), ff_w2, r(ff_b2), r(n2_g), r(n2_b))
    return out.reshape(D, N, DMODEL)


# speedup vs baseline: 1.1031x; 1.0267x over previous
import functools

import jax
import jax.numpy as jnp
from jax import lax
from jax.experimental import pallas as pl
from jax.experimental.pallas import tpu as pltpu
from jax.experimental.pallas import tpu_sc as plsc

DMODEL = 1024
H = DMODEL // 2
DQK = 64
HEADS = 16
HD = HEADS * DQK
FF = 2048
NB = 64
PEN = 100000.0
D = 2
N = 4096
T = D * N
CH = 128
BC = 8
NCHUNK = T // CH

_F32 = jnp.float32


HBLK = 1024
NHSTEP = T // HBLK


def _hash_sort_body(x_ref, wqk_ref, bqk_ref, hm_ref, x2_ref, p_ref, h_acc):
    i = pl.program_id(0)

    @pl.when(i < NHSTEP)
    def _hash():
        x2 = x_ref[...]
        x2_ref[...] = x2
        qk = jnp.dot(x2, wqk_ref[...],
                     preferred_element_type=_F32) + bqk_ref[...]
        proj = jnp.dot(qk, hm_ref[...], preferred_element_type=_F32)
        m = jnp.maximum(jnp.max(proj, axis=1, keepdims=True),
                        jnp.max(-proj, axis=1, keepdims=True))
        io = lax.broadcasted_iota(jnp.int32, (HBLK, NB // 2), 1)
        big = jnp.int32(2 * NB)
        i1 = jnp.min(jnp.where(proj == m, io, big), axis=1, keepdims=True)
        i2 = jnp.min(jnp.where(-proj == m, io + (NB // 2), big), axis=1,
                     keepdims=True)
        h_acc[pl.ds(i * HBLK, HBLK), :] = jnp.minimum(i1, i2)

    @pl.when(i >= NHSTEP)
    def _sort():
        b = i - NHSTEP
        h = h_acc[pl.ds(b * N, N), :]
        buckets = lax.broadcasted_iota(jnp.int32, (N, NB), 1)
        onehot = (h == buckets).astype(_F32)
        counts = jnp.sum(onehot, axis=0, keepdims=True)
        r0 = lax.broadcasted_iota(jnp.int32, (NB, NB), 0)
        c0 = lax.broadcasted_iota(jnp.int32, (NB, NB), 1)
        lt = (r0 < c0).astype(_F32)
        starts = jnp.dot(counts, lt, preferred_element_type=_F32)
        tb = 512
        nblk = N // tb
        r1 = lax.broadcasted_iota(jnp.int32, (tb, tb), 0)
        c1 = lax.broadcasted_iota(jnp.int32, (tb, tb), 1)
        lstrict = (c1 < r1).astype(_F32)
        run = starts
        for j in range(nblk):
            ob = lax.slice(onehot, (j * tb, 0), ((j + 1) * tb, NB))
            cb = jnp.dot(lstrict, ob, preferred_element_type=_F32) + run
            pb = jnp.sum(cb * ob, axis=1, keepdims=True)
            run = run + jnp.sum(ob, axis=0, keepdims=True)
            p_ref[0, j, :] = pb[:, 0].astype(jnp.int32) + b * N


def _hash_and_sort(xf, wqk, bqk, hm):
    nstep = NHSTEP + D
    x2f, p = pl.pallas_call(
        _hash_sort_body,
        grid=(nstep,),
        in_specs=[
            pl.BlockSpec((HBLK, H), lambda i: (jnp.minimum(i, NHSTEP - 1), 1)),
            pl.BlockSpec((H, HD), lambda i: (0, 0)),
            pl.BlockSpec((1, HD), lambda i: (0, 0)),
            pl.BlockSpec((HD, NB // 2), lambda i: (0, 0)),
        ],
        out_specs=[
            pl.BlockSpec((HBLK, H), lambda i: (jnp.minimum(i, NHSTEP - 1), 0)),
            pl.BlockSpec((1, N // 512, 512),
                         lambda i: (jnp.maximum(i - NHSTEP, 0), 0, 0)),
        ],
        out_shape=[
            jax.ShapeDtypeStruct((T, H), _F32),
            jax.ShapeDtypeStruct((D, N // 512, 512), jnp.int32),
        ],
        scratch_shapes=[pltpu.VMEM((T, 1), jnp.int32)],
        compiler_params=pltpu.CompilerParams(
            dimension_semantics=("arbitrary",)),
    )(xf, wqk, bqk, hm)
    return x2f, p.reshape(T)


RC = 64


def _sc_permute(x2f, pos2d):
    info = plsc.get_sparse_core_info()
    nw = info.num_cores * info.num_subcores
    per_w = T // nw
    nchunk = per_w // RC
    mesh = plsc.VectorSubcoreMesh(core_axis_name="c", subcore_axis_name="s")

    @functools.partial(
        pl.kernel,
        mesh=mesh,
        out_type=jax.ShapeDtypeStruct((T, H), _F32),
        scratch_types=[
            pltpu.VMEM((nchunk, RC), jnp.int32),
            pltpu.VMEM((RC, H), _F32),
            pltpu.VMEM((RC, H), _F32),
            pltpu.VMEM((RC, H), _F32),
            pltpu.SemaphoreType.DMA,
            pltpu.SemaphoreType.DMA,
            pltpu.SemaphoreType.DMA,
            pltpu.SemaphoreType.DMA,
            pltpu.SemaphoreType.DMA,
            pltpu.SemaphoreType.DMA,
        ],
    )
    def scatter_k(tab_hbm, idx_hbm, out_hbm, idx_v, b0, b1, b2,
                  l0, l1, l2, s0, s1, s2):
        wid = lax.axis_index("s") * info.num_cores + lax.axis_index("c")
        base = wid * per_w
        pltpu.sync_copy(idx_hbm.at[pl.ds(wid * nchunk, nchunk)], idx_v)
        bufs = [b0, b1, b2]
        lsems = [l0, l1, l2]
        ssems = [s0, s1, s2]

        def load(j, b):
            return pltpu.async_copy(tab_hbm.at[pl.ds(base + j * RC, RC)],
                                    bufs[b], lsems[b])

        def scat(j, b):
            return pltpu.async_copy(bufs[b], out_hbm.at[idx_v.at[j]],
                                    ssems[b])

        loads = {0: load(0, 0), 1: load(1, 1)}
        scats = {}
        for j in range(nchunk):
            b = j % 3
            loads[j].wait()
            scats[j] = scat(j, b)
            nj = j + 2
            if nj < nchunk:
                if nj - 3 >= 0:
                    scats[nj - 3].wait()
                loads[nj] = load(nj, nj % 3)
        for j in range(max(nchunk - 3, 1), nchunk):
            scats[j].wait()

    return scatter_k(x2f, pos2d)
def _tail_body(x_ref, x2g_ref, wqk_ref, bqk_ref, wv_ref, bv_ref,
               wu_ref, bu_ref, n1g_ref, n1b_ref,
               w1_ref, b1_ref, w2_ref, b2_ref, n2g_ref, n2b_ref, o_ref):
    x2g = x2g_ref[...]
    qkf = jnp.dot(x2g, wqk_ref[...], preferred_element_type=_F32) + bqk_ref[...]
    valsf = jnp.dot(x2g, wv_ref[...], preferred_element_type=_F32) + bv_ref[...]
    rr = lax.broadcasted_iota(jnp.int32, (CH, CH), 0)
    cc = lax.broadcasted_iota(jnp.int32, (CH, CH), 1)
    parts = []
    for sub in range(BC):
        q = lax.slice(qkf, (sub * CH, 0), ((sub + 1) * CH, HD))
        v = lax.slice(valsf, (sub * CH, 0), ((sub + 1) * CH, HD))
        s = lax.dot_general(q, q, (((1,), (1,)), ((), ())),
                            preferred_element_type=_F32) * _F32(1.0 / 8.0)
        s = jnp.where(rr == cc, s / _F32(PEN), s)
        parts.append(jnp.dot(s, v, preferred_element_type=_F32))
    attn = jnp.concatenate(parts, axis=0)
    ua = jnp.dot(attn, wu_ref[...], preferred_element_type=_F32) + bu_ref[...]

    x1 = x_ref[:, :H]
    x2 = x_ref[:, H:]
    t1 = x1 + ua
    mu = jnp.mean(t1, axis=1, keepdims=True)
    var = jnp.mean((t1 - mu) ** 2, axis=1, keepdims=True)
    y1 = (t1 - mu) / jnp.sqrt(var + _F32(1e-5)) * n1g_ref[...] + n1b_ref[...]

    ff = jnp.maximum(jnp.dot(y1, w1_ref[...], preferred_element_type=_F32)
                     + b1_ref[...], _F32(0.0))
    ffo = jnp.dot(ff, w2_ref[...], preferred_element_type=_F32) + b2_ref[...]
    t2 = x2 + ffo
    mu2 = jnp.mean(t2, axis=1, keepdims=True)
    var2 = jnp.mean((t2 - mu2) ** 2, axis=1, keepdims=True)
    y2 = (t2 - mu2) / jnp.sqrt(var2 + _F32(1e-5)) * n2g_ref[...] + n2b_ref[...]

    o_ref[:, :H] = y1
    o_ref[:, H:] = y2


def _attention_tail(xf, x2g, wqk, bqk, wv, bv, wu, bu, n1g, n1b,
                    w1, b1, w2, b2, n2g, n2b):
    res = lambda shape: pl.BlockSpec(shape, lambda c: tuple(0 for _ in shape))
    return pl.pallas_call(
        _tail_body,
        grid=(NCHUNK // BC,),
        in_specs=[
            pl.BlockSpec((BC * CH, DMODEL), lambda c: (c, 0)),
            pl.BlockSpec((BC * CH, H), lambda c: (c, 0)),
            res((H, HD)), res((1, HD)),
            res((H, HD)), res((1, HD)),
            res((HD, H)), res((1, H)),
            res((1, H)), res((1, H)),
            res((H, FF)), res((1, FF)),
            res((FF, H)), res((1, H)),
            res((1, H)), res((1, H)),
        ],
        out_specs=pl.BlockSpec((BC * CH, DMODEL), lambda c: (c, 0)),
        out_shape=jax.ShapeDtypeStruct((T, DMODEL), _F32),
    )(xf, x2g, wqk, bqk, wv, bv, wu, bu, n1g, n1b, w1, b1, w2, b2, n2g, n2b)


def kernel(x, Wqk_w, Wqk_b, Wv_w, Wv_b, unify_w, unify_b, n1_g, n1_b,
           ff_w1, ff_b1, ff_w2, ff_b2, n2_g, n2_b, hashM):
    xf = x.reshape(T, DMODEL)
    r = lambda v: v.reshape(1, -1)
    x2f, pos = _hash_and_sort(xf, Wqk_w, r(Wqk_b), hashM)
    x2g = _sc_permute(x2f, pos.reshape(T // RC, RC))
    out = _attention_tail(xf, x2g, Wqk_w, r(Wqk_b), Wv_w, r(Wv_b),
                          unify_w, r(unify_b), r(n1_g), r(n1_b),
                          ff_w1, r(ff_b1), ff_w2, r(ff_b2), r(n2_g), r(n2_b))
    return out.reshape(D, N, DMODEL)
